# baseline (speedup 1.0000x reference)
import functools

import jax
import jax.numpy as jnp
from jax import lax
from jax.experimental import pallas as pl
from jax.experimental.pallas import tpu as pltpu
from jax.experimental.pallas import tpu_sc as plsc

NC = 2
NS = 16
NW = NC * NS
AW = 128


def _pick_chunk(epw):
    for c in range(128, 7, -8):
        if epw % c == 0:
            return c
    raise ValueError(f"no valid chunk for {epw}")


def _precompute(nf, w_src, w_dst, bn):
    n, nd = nf.shape
    h = w_src.shape[1]

    def body(nf_ref, ws_ref, wd_ref, o1_ref, o2_ref):
        x = nf_ref[...]
        o1_ref[...] = jnp.dot(x, ws_ref[...], preferred_element_type=jnp.float32)
        o2_ref[...] = jnp.dot(x, wd_ref[...], preferred_element_type=jnp.float32)

    return pl.pallas_call(
        body,
        grid=(n // bn,),
        in_specs=[
            pl.BlockSpec((bn, nd), lambda i: (i, 0)),
            pl.BlockSpec((nd, h), lambda i: (0, 0)),
            pl.BlockSpec((nd, h), lambda i: (0, 0)),
        ],
        out_specs=[
            pl.BlockSpec((bn, h), lambda i: (i, 0)),
            pl.BlockSpec((bn, h), lambda i: (i, 0)),
        ],
        out_shape=[
            jax.ShapeDtypeStruct((n, h), jnp.float32),
            jax.ShapeDtypeStruct((n, h), jnp.float32),
        ],
    )(nf, w_src, w_dst)


def _sc_gather(psrc, pdst, src, dst, ch):
    n, h = psrc.shape
    nw, k, ch2 = src.shape
    assert ch2 == ch
    e = nw * k * ch
    epw = e // NW
    mesh = plsc.VectorSubcoreMesh(core_axis_name="c", subcore_axis_name="s")

    @functools.partial(
        pl.kernel,
        out_type=jax.ShapeDtypeStruct((e, h), jnp.float32),
        mesh=mesh,
        scratch_types=[
            pltpu.VMEM((k, ch), jnp.int32),
            pltpu.VMEM((k, ch), jnp.int32),
            pltpu.VMEM((ch, h), jnp.float32),
            pltpu.VMEM((ch, h), jnp.float32),
            pltpu.VMEM((ch, h), jnp.float32),
            pltpu.VMEM((ch, h), jnp.float32),
            pltpu.VMEM((ch, h), jnp.float32),
            pltpu.VMEM((ch, h), jnp.float32),
            pltpu.SemaphoreType.DMA,
            pltpu.SemaphoreType.DMA,
            pltpu.SemaphoreType.DMA,
            pltpu.SemaphoreType.DMA,
            pltpu.SemaphoreType.DMA,
            pltpu.SemaphoreType.DMA,
        ],
    )
    def gather_k(psrc_hbm, pdst_hbm, src_hbm, dst_hbm, g_hbm,
                 si_v, di_v, a0, a1, b0, b1, o0, o1,
                 sa0, sa1, sb0, sb1, sw0, sw1):
        wid = lax.axis_index("s") * NC + lax.axis_index("c")
        pltpu.sync_copy(src_hbm.at[wid], si_v)
        pltpu.sync_copy(dst_hbm.at[wid], di_v)

        def fire(j, a, b, sa, sb):
            pltpu.async_copy(psrc_hbm.at[si_v.at[j]], a, sa)
            pltpu.async_copy(pdst_hbm.at[di_v.at[j]], b, sb)

        def stage(j, a, b, o, sa, sb, sw, first):
            pltpu.make_async_copy(
                psrc_hbm.at[si_v.at[j]], a, sa).wait()
            pltpu.make_async_copy(
                pdst_hbm.at[di_v.at[j]], b, sb).wait()
            if not first:
                pltpu.make_async_copy(o, g_hbm.at[pl.ds(0, ch)], sw).wait()

            def row(i, carry):
                for g in range(h // 16):
                    o[i, pl.ds(16 * g, 16)] = (a[i, pl.ds(16 * g, 16)]
                                               + b[i, pl.ds(16 * g, 16)])
                return carry

            lax.fori_loop(0, ch, row, 0)

            @pl.when(jnp.int32(j) + 2 < k)
            def _prefetch():
                fire(jnp.int32(j) + 2, a, b, sa, sb)

            pltpu.async_copy(o, g_hbm.at[pl.ds(wid * epw + j * ch, ch)], sw)

        fire(0, a0, b0, sa0, sb0)
        fire(1, a1, b1, sa1, sb1)
        stage(0, a0, b0, o0, sa0, sb0, sw0, True)
        stage(1, a1, b1, o1, sa1, sb1, sw1, True)

        def pair(i2, carry):
            stage(2 * i2, a0, b0, o0, sa0, sb0, sw0, False)
            stage(2 * i2 + 1, a1, b1, o1, sa1, sb1, sw1, False)
            return carry

        lax.fori_loop(1, k // 2, pair, 0)
        if k % 2 == 1:
            stage(k - 1, a0, b0, o0, sa0, sb0, sw0, False)

        pltpu.make_async_copy(o0, g_hbm.at[pl.ds(0, ch)], sw0).wait()
        pltpu.make_async_copy(o1, g_hbm.at[pl.ds(0, ch)], sw1).wait()

    return gather_k(psrc, pdst, src, dst)


def _edge_mlp(ef, g, w1e, b1, w2, b2, be):
    e, ed = ef.shape
    h = w1e.shape[1]

    def body(ef_ref, g_ref, w1_ref, b1_ref, w2_ref, b2_ref, out_ref):
        x = ef_ref[...]
        pre = (g_ref[...] + b1_ref[...]
               + jnp.dot(x, w1_ref[...], preferred_element_type=jnp.float32))
        hid = pre * jax.nn.sigmoid(pre)
        out_ref[...] = x + b2_ref[...] + jnp.dot(
            hid, w2_ref[...], preferred_element_type=jnp.float32)

    return pl.pallas_call(
        body,
        grid=(e // be,),
        in_specs=[
            pl.BlockSpec((be, ed), lambda i: (i, 0)),
            pl.BlockSpec((be, h), lambda i: (i, 0)),
            pl.BlockSpec((ed, h), lambda i: (0, 0)),
            pl.BlockSpec((1, h), lambda i: (0, 0)),
            pl.BlockSpec((h, ed), lambda i: (0, 0)),
            pl.BlockSpec((1, ed), lambda i: (0, 0)),
        ],
        out_specs=pl.BlockSpec((be, ed), lambda i: (i, 0)),
        out_shape=jax.ShapeDtypeStruct((e, ed), jnp.float32),
    )(ef, g, w1e, b1, w2, b2)


def _sc_scatter(eo, dst, zeros_tab, n, ch):
    e, ed = eo.shape
    epw = e // NW
    k = epw // ch
    nrs = 1000
    ndt = n // nrs
    mesh = plsc.VectorSubcoreMesh(core_axis_name="c", subcore_axis_name="s")

    @functools.partial(
        pl.kernel,
        out_type=jax.ShapeDtypeStruct((NC, n, AW), jnp.float32),
        mesh=mesh,
        scratch_types=[
            pltpu.VMEM((k, ch), jnp.int32),
            pltpu.VMEM((ch, ed), jnp.float32),
            pltpu.VMEM((ch, ed), jnp.float32),
            pltpu.VMEM((ch, AW), jnp.float32),
            pltpu.SemaphoreType.DMA,
            pltpu.SemaphoreType.DMA,
            pltpu.SemaphoreType.DMA,
            pltpu.VMEM_SHARED((n, AW), jnp.float32),
        ],
    )
    def scatter_k(eo_hbm, dst_hbm, z_hbm, part_hbm,
                  di2, in0, in1, pay, sl0, sl1, ss, tab_sh):
        cid = lax.axis_index("c")
        sid = lax.axis_index("s")
        wid = sid * NC + cid
        rb = sid * nrs

        @pl.when(sid < ndt)
        def _init():
            pltpu.sync_copy(z_hbm, tab_sh.at[pl.ds(rb, nrs)])

        pltpu.sync_copy(dst_hbm.at[wid], di2)

        pltpu.sync_copy(z_hbm.at[pl.ds(0, ch)], pay)
        cnt_vec = jnp.where(lax.iota(jnp.int32, 16) == 0,
                            jnp.float32(1.0), jnp.float32(0.0))

        def prow(i, carry):
            pay[i, pl.ds(16, 16)] = cnt_vec
            return carry

        lax.fori_loop(0, ch, prow, 0)
        plsc.subcore_barrier()

        def load(j, in_v, sl):
            pltpu.async_copy(
                eo_hbm.at[pl.ds(wid * epw + j * ch, ch)], in_v, sl)

        def stage(j, in_v, sl, first):
            pltpu.make_async_copy(
                eo_hbm.at[pl.ds(0, ch)], in_v, sl).wait()
            if not first:
                pltpu.make_async_copy(pay, tab_sh.at[di2.at[0]], ss).wait()

            def crow(i, carry2):
                pay[i, pl.ds(0, 16)] = in_v[i, :]
                return carry2

            lax.fori_loop(0, ch, crow, 0)

            @pl.when(jnp.int32(j) + 2 < k)
            def _prefetch():
                load(jnp.int32(j) + 2, in_v, sl)

            pltpu.async_copy(pay, tab_sh.at[di2.at[j]], ss, add=True)

        load(0, in0, sl0)
        load(1, in1, sl1)
        stage(0, in0, sl0, True)
        stage(1, in1, sl1, False)

        def pair(i2, carry):
            stage(2 * i2, in0, sl0, False)
            stage(2 * i2 + 1, in1, sl1, False)
            return carry

        lax.fori_loop(1, k // 2, pair, 0)
        if k % 2 == 1:
            stage(k - 1, in0, sl0, False)

        pltpu.make_async_copy(pay, tab_sh.at[di2.at[0]], ss).wait()
        plsc.subcore_barrier()

        @pl.when(sid < ndt)
        def _drain():
            pltpu.sync_copy(tab_sh.at[pl.ds(rb, nrs)],
                            part_hbm.at[cid, pl.ds(rb, nrs)])

    return scatter_k(eo, dst, zeros_tab)


def _node_mlp(nf, p0, p1, p2, p3, w1a, w1b, b1, w2, b2, bn):
    n, nd = nf.shape
    ed = w1b.shape[0]
    h = w1a.shape[1]

    def body(nf_ref, p0_ref, p1_ref, p2_ref, p3_ref,
             w1a_ref, w1b_ref, b1_ref, w2_ref, b2_ref, out_ref):
        x = nf_ref[...]
        tab = (p0_ref[...] + p1_ref[...]) + (p2_ref[...] + p3_ref[...])
        agg = tab[:, :ed]
        cts = tab[:, ed:ed + 1]
        mean = agg / jnp.maximum(cts, 1.0)
        pre = (jnp.dot(x, w1a_ref[...], preferred_element_type=jnp.float32)
               + jnp.dot(mean, w1b_ref[...], preferred_element_type=jnp.float32)
               + b1_ref[...])
        hid = pre * jax.nn.sigmoid(pre)
        out_ref[...] = x + b2_ref[...] + jnp.dot(
            hid, w2_ref[...], preferred_element_type=jnp.float32)

    return pl.pallas_call(
        body,
        grid=(n // bn,),
        in_specs=[
            pl.BlockSpec((bn, nd), lambda i: (i, 0)),
            pl.BlockSpec((bn, AW), lambda i: (i, 0)),
            pl.BlockSpec((bn, AW), lambda i: (i, 0)),
            pl.BlockSpec((bn, AW), lambda i: (i, 0)),
            pl.BlockSpec((bn, AW), lambda i: (i, 0)),
            pl.BlockSpec((nd, h), lambda i: (0, 0)),
            pl.BlockSpec((ed, h), lambda i: (0, 0)),
            pl.BlockSpec((1, h), lambda i: (0, 0)),
            pl.BlockSpec((h, nd), lambda i: (0, 0)),
            pl.BlockSpec((1, nd), lambda i: (0, 0)),
        ],
        out_specs=pl.BlockSpec((bn, nd), lambda i: (i, 0)),
        out_shape=jax.ShapeDtypeStruct((n, nd), jnp.float32),
    )(nf, p0, p1, p2, p3, w1a, w1b, b1, w2, b2)


def kernel(node_feat, edge_feat, edge_index, We1, be1, We2, be2,
           Wn1, bn1, Wn2, bn2):
    n, nd = node_feat.shape
    e, ed = edge_feat.shape
    h = We1.shape[1]

    src = edge_index[0].astype(jnp.int32)
    dst = edge_index[1].astype(jnp.int32)

    eh = e // 2
    epw = eh // NW
    ch = _pick_chunk(epw)
    k = epw // ch

    psrc, pdst = _precompute(node_feat, We1[ed:ed + nd], We1[ed + nd:], bn=2000)
    zeros_tab = jnp.zeros((1000, AW), jnp.float32)

    eo_halves = []
    parts = []
    for half in range(2):
        s_h = lax.slice_in_dim(src, half * eh, (half + 1) * eh)
        d_h = lax.slice_in_dim(dst, half * eh, (half + 1) * eh)
        ef_h = lax.slice_in_dim(edge_feat, half * eh, (half + 1) * eh)
        g12 = _sc_gather(psrc, pdst, s_h.reshape(NW, k, ch),
                         d_h.reshape(NW, k, ch), ch)
        eo_h = _edge_mlp(ef_h, g12, We1[:ed],
                         be1.reshape(1, h), We2, be2.reshape(1, ed), be=3200)
        eo_halves.append(eo_h)
        parts.append(_sc_scatter(eo_h, d_h.reshape(NW, k, ch),
                                 zeros_tab, n, ch))

    edge_out = jnp.concatenate(eo_halves, axis=0)
    node_out = _node_mlp(node_feat, parts[0][0], parts[0][1],
                         parts[1][0], parts[1][1],
                         Wn1[:nd], Wn1[nd:], bn1.reshape(1, nd),
                         Wn2, bn2.reshape(1, nd), bn=2000)
    return (node_out, edge_out)


# speedup vs baseline: 1.0468x; 1.0468x over previous
import functools

import jax
import jax.numpy as jnp
from jax import lax
from jax.experimental import pallas as pl
from jax.experimental.pallas import tpu as pltpu
from jax.experimental.pallas import tpu_sc as plsc

NC = 2
NS = 16
NW = NC * NS
AW = 128


def _pick_chunk(epw):
    for c in range(128, 7, -8):
        if epw % c == 0:
            return c
    raise ValueError(f"no valid chunk for {epw}")


def _precompute(nf, w_src, w_dst, bn):
    n, nd = nf.shape
    h = w_src.shape[1]

    def body(nf_ref, ws_ref, wd_ref, o1_ref, o2_ref):
        x = nf_ref[...]
        o1_ref[...] = jnp.dot(x, ws_ref[...], preferred_element_type=jnp.float32)
        o2_ref[...] = jnp.dot(x, wd_ref[...], preferred_element_type=jnp.float32)

    return pl.pallas_call(
        body,
        grid=(n // bn,),
        in_specs=[
            pl.BlockSpec((bn, nd), lambda i: (i, 0)),
            pl.BlockSpec((nd, h), lambda i: (0, 0)),
            pl.BlockSpec((nd, h), lambda i: (0, 0)),
        ],
        out_specs=[
            pl.BlockSpec((bn, h), lambda i: (i, 0)),
            pl.BlockSpec((bn, h), lambda i: (i, 0)),
        ],
        out_shape=[
            jax.ShapeDtypeStruct((n, h), jnp.float32),
            jax.ShapeDtypeStruct((n, h), jnp.float32),
        ],
    )(nf, w_src, w_dst)


def _sc_gather(psrc, pdst, src, dst, ch):
    n, h = psrc.shape
    nw, k, ch2 = src.shape
    assert ch2 == ch
    e = nw * k * ch
    epw = e // NW
    mesh = plsc.VectorSubcoreMesh(core_axis_name="c", subcore_axis_name="s")

    @functools.partial(
        pl.kernel,
        out_type=jax.ShapeDtypeStruct((e, h), jnp.float32),
        mesh=mesh,
        scratch_types=[
            pltpu.VMEM((k, ch), jnp.int32),
            pltpu.VMEM((k, ch), jnp.int32),
            pltpu.VMEM((ch, h), jnp.float32),
            pltpu.VMEM((ch, h), jnp.float32),
            pltpu.VMEM((ch, h), jnp.float32),
            pltpu.VMEM((ch, h), jnp.float32),
            pltpu.VMEM((ch, h), jnp.float32),
            pltpu.VMEM((ch, h), jnp.float32),
            pltpu.SemaphoreType.DMA,
            pltpu.SemaphoreType.DMA,
            pltpu.SemaphoreType.DMA,
            pltpu.SemaphoreType.DMA,
            pltpu.SemaphoreType.DMA,
            pltpu.SemaphoreType.DMA,
        ],
    )
    def gather_k(psrc_hbm, pdst_hbm, src_hbm, dst_hbm, g_hbm,
                 si_v, di_v, a0, a1, b0, b1, o0, o1,
                 sa0, sa1, sb0, sb1, sw0, sw1):
        wid = lax.axis_index("s") * NC + lax.axis_index("c")
        pltpu.sync_copy(src_hbm.at[wid], si_v)
        pltpu.sync_copy(dst_hbm.at[wid], di_v)

        def fire(j, a, b, sa, sb):
            pltpu.async_copy(psrc_hbm.at[si_v.at[j]], a, sa)
            pltpu.async_copy(pdst_hbm.at[di_v.at[j]], b, sb)

        def stage(j, a, b, o, sa, sb, sw, first):
            pltpu.make_async_copy(
                psrc_hbm.at[si_v.at[j]], a, sa).wait()
            pltpu.make_async_copy(
                pdst_hbm.at[di_v.at[j]], b, sb).wait()
            if not first:
                pltpu.make_async_copy(o, g_hbm.at[pl.ds(0, ch)], sw).wait()

            def row(i, carry):
                for g in range(h // 16):
                    o[i, pl.ds(16 * g, 16)] = (a[i, pl.ds(16 * g, 16)]
                                               + b[i, pl.ds(16 * g, 16)])
                return carry

            lax.fori_loop(0, ch, row, 0)

            @pl.when(jnp.int32(j) + 2 < k)
            def _prefetch():
                fire(jnp.int32(j) + 2, a, b, sa, sb)

            pltpu.async_copy(o, g_hbm.at[pl.ds(wid * epw + j * ch, ch)], sw)

        fire(0, a0, b0, sa0, sb0)
        fire(1, a1, b1, sa1, sb1)
        stage(0, a0, b0, o0, sa0, sb0, sw0, True)
        stage(1, a1, b1, o1, sa1, sb1, sw1, True)

        def pair(i2, carry):
            stage(2 * i2, a0, b0, o0, sa0, sb0, sw0, False)
            stage(2 * i2 + 1, a1, b1, o1, sa1, sb1, sw1, False)
            return carry

        lax.fori_loop(1, k // 2, pair, 0)
        if k % 2 == 1:
            stage(k - 1, a0, b0, o0, sa0, sb0, sw0, False)

        pltpu.make_async_copy(o0, g_hbm.at[pl.ds(0, ch)], sw0).wait()
        pltpu.make_async_copy(o1, g_hbm.at[pl.ds(0, ch)], sw1).wait()

    return gather_k(psrc, pdst, src, dst)


def _edge_mlp(ef, g, w1e, b1, w2, b2, be):
    e, ed = ef.shape
    h = w1e.shape[1]

    def body(ef_ref, g_ref, w1_ref, b1_ref, w2_ref, b2_ref, out_ref):
        x = ef_ref[...]
        pre = (g_ref[...] + b1_ref[...]
               + jnp.dot(x, w1_ref[...], preferred_element_type=jnp.float32))
        hid = pre * jax.nn.sigmoid(pre)
        out_ref[...] = x + b2_ref[...] + jnp.dot(
            hid, w2_ref[...], preferred_element_type=jnp.float32)

    return pl.pallas_call(
        body,
        grid=(e // be,),
        in_specs=[
            pl.BlockSpec((be, ed), lambda i: (i, 0)),
            pl.BlockSpec((be, h), lambda i: (i, 0)),
            pl.BlockSpec((ed, h), lambda i: (0, 0)),
            pl.BlockSpec((1, h), lambda i: (0, 0)),
            pl.BlockSpec((h, ed), lambda i: (0, 0)),
            pl.BlockSpec((1, ed), lambda i: (0, 0)),
        ],
        out_specs=pl.BlockSpec((be, ed), lambda i: (i, 0)),
        out_shape=jax.ShapeDtypeStruct((e, ed), jnp.float32),
    )(ef, g, w1e, b1, w2, b2)


def _sc_scatter(eo, dst, zeros_tab, n, ch):
    e, ed = eo.shape
    epw = e // NW
    k = epw // ch
    nrs = 1000
    ndt = n // nrs
    mesh = plsc.VectorSubcoreMesh(core_axis_name="c", subcore_axis_name="s")

    @functools.partial(
        pl.kernel,
        out_type=jax.ShapeDtypeStruct((NC, n, AW), jnp.float32),
        mesh=mesh,
        scratch_types=[
            pltpu.VMEM((k, ch), jnp.int32),
            pltpu.VMEM((ch, ed), jnp.float32),
            pltpu.VMEM((ch, ed), jnp.float32),
            pltpu.VMEM((ch, AW), jnp.float32),
            pltpu.SemaphoreType.DMA,
            pltpu.SemaphoreType.DMA,
            pltpu.SemaphoreType.DMA,
            pltpu.VMEM_SHARED((n, AW), jnp.float32),
        ],
    )
    def scatter_k(eo_hbm, dst_hbm, z_hbm, part_hbm,
                  di2, in0, in1, pay, sl0, sl1, ss, tab_sh):
        cid = lax.axis_index("c")
        sid = lax.axis_index("s")
        wid = sid * NC + cid
        rb = sid * nrs

        @pl.when(sid < ndt)
        def _init():
            pltpu.sync_copy(z_hbm, tab_sh.at[pl.ds(rb, nrs)])

        pltpu.sync_copy(dst_hbm.at[wid], di2)

        pltpu.sync_copy(z_hbm.at[pl.ds(0, ch)], pay)
        cnt_vec = jnp.where(lax.iota(jnp.int32, 16) == 0,
                            jnp.float32(1.0), jnp.float32(0.0))

        def prow(i, carry):
            pay[i, pl.ds(16, 16)] = cnt_vec
            return carry

        lax.fori_loop(0, ch, prow, 0)
        plsc.subcore_barrier()

        def load(j, in_v, sl):
            pltpu.async_copy(
                eo_hbm.at[pl.ds(wid * epw + j * ch, ch)], in_v, sl)

        def stage(j, in_v, sl, first):
            pltpu.make_async_copy(
                eo_hbm.at[pl.ds(0, ch)], in_v, sl).wait()
            if not first:
                pltpu.make_async_copy(pay, tab_sh.at[di2.at[0]], ss).wait()

            def crow(i, carry2):
                pay[i, pl.ds(0, 16)] = in_v[i, :]
                return carry2

            lax.fori_loop(0, ch, crow, 0)

            @pl.when(jnp.int32(j) + 2 < k)
            def _prefetch():
                load(jnp.int32(j) + 2, in_v, sl)

            pltpu.async_copy(pay, tab_sh.at[di2.at[j]], ss, add=True)

        load(0, in0, sl0)
        load(1, in1, sl1)
        stage(0, in0, sl0, True)
        stage(1, in1, sl1, False)

        def pair(i2, carry):
            stage(2 * i2, in0, sl0, False)
            stage(2 * i2 + 1, in1, sl1, False)
            return carry

        lax.fori_loop(1, k // 2, pair, 0)
        if k % 2 == 1:
            stage(k - 1, in0, sl0, False)

        pltpu.make_async_copy(pay, tab_sh.at[di2.at[0]], ss).wait()
        plsc.subcore_barrier()

        @pl.when(sid < ndt)
        def _drain():
            pltpu.sync_copy(tab_sh.at[pl.ds(rb, nrs)],
                            part_hbm.at[cid, pl.ds(rb, nrs)])

    return scatter_k(eo, dst, zeros_tab)


def _node_mlp(nf, p0, p1, w1a, w1b, b1, w2, b2, bn):
    n, nd = nf.shape
    ed = w1b.shape[0]
    h = w1a.shape[1]

    def body(nf_ref, p0_ref, p1_ref,
             w1a_ref, w1b_ref, b1_ref, w2_ref, b2_ref, out_ref):
        x = nf_ref[...]
        tab = p0_ref[...] + p1_ref[...]
        agg = tab[:, :ed]
        cts = tab[:, ed:ed + 1]
        mean = agg / jnp.maximum(cts, 1.0)
        pre = (jnp.dot(x, w1a_ref[...], preferred_element_type=jnp.float32)
               + jnp.dot(mean, w1b_ref[...], preferred_element_type=jnp.float32)
               + b1_ref[...])
        hid = pre * jax.nn.sigmoid(pre)
        out_ref[...] = x + b2_ref[...] + jnp.dot(
            hid, w2_ref[...], preferred_element_type=jnp.float32)

    return pl.pallas_call(
        body,
        grid=(n // bn,),
        in_specs=[
            pl.BlockSpec((bn, nd), lambda i: (i, 0)),
            pl.BlockSpec((bn, AW), lambda i: (i, 0)),
            pl.BlockSpec((bn, AW), lambda i: (i, 0)),
            pl.BlockSpec((nd, h), lambda i: (0, 0)),
            pl.BlockSpec((ed, h), lambda i: (0, 0)),
            pl.BlockSpec((1, h), lambda i: (0, 0)),
            pl.BlockSpec((h, nd), lambda i: (0, 0)),
            pl.BlockSpec((1, nd), lambda i: (0, 0)),
        ],
        out_specs=pl.BlockSpec((bn, nd), lambda i: (i, 0)),
        out_shape=jax.ShapeDtypeStruct((n, nd), jnp.float32),
    )(nf, p0, p1, w1a, w1b, b1, w2, b2)


def kernel(node_feat, edge_feat, edge_index, We1, be1, We2, be2,
           Wn1, bn1, Wn2, bn2):
    n, nd = node_feat.shape
    e, ed = edge_feat.shape
    h = We1.shape[1]

    src = edge_index[0].astype(jnp.int32)
    dst = edge_index[1].astype(jnp.int32)
    epw = e // NW
    ch = _pick_chunk(epw)
    k = epw // ch

    psrc, pdst = _precompute(node_feat, We1[ed:ed + nd], We1[ed + nd:], bn=2000)
    g12 = _sc_gather(psrc, pdst, src.reshape(NW, k, ch),
                     dst.reshape(NW, k, ch), ch)

    edge_out = _edge_mlp(edge_feat, g12, We1[:ed],
                         be1.reshape(1, h), We2, be2.reshape(1, ed), be=3200)

    zeros_tab = jnp.zeros((1000, AW), jnp.float32)
    parts = _sc_scatter(edge_out, dst.reshape(NW, k, ch), zeros_tab, n, ch)

    node_out = _node_mlp(node_feat, parts[0], parts[1],
                         Wn1[:nd], Wn1[nd:], bn1.reshape(1, nd),
                         Wn2, bn2.reshape(1, nd), bn=2000)
    return (node_out, edge_out)
